# Initial kernel scaffold; baseline (speedup 1.0000x reference)
import jax
import jax.numpy as jnp
from jax.experimental import pallas as pl


def kernel(x_adresse, x_batiment, x_parcelle, ei_ab, ei_bp, ei_pa, Wl_ab, bl_ab, Wr_ab, Wl_bp, bl_bp, Wr_bp, Wl_pa, bl_pa, Wr_pa, lin_W, lin_b):
    raise NotImplementedError("write your pallas kernel here")


# speedup vs baseline: 2.7159x; 2.7159x over previous
import jax
import jax.numpy as jnp
from jax import lax
from jax.experimental import pallas as pl
from jax.experimental.pallas import tpu as pltpu
from jax.experimental.pallas import tpu_sc as plsc

N = 10000
D = 256
H = 256
O = 128
E = 160000
HALF = D // 2
NC = 2
NS = 16
CH = 128
EPT = 10240
NK = EPT // CH
NKH = NK // NC
EP = NS * EPT
NP = 10240
RPT = NP // NS
CW = 16
DUMP = NP - CW


def _sum_body(xc_a, xc_b, xc_p, s_ab, d_ab, s_bp, d_bp, s_pa, d_pa, zf_h,
              sums, acc, isrc, idst, gbuf, gsem):
    c = lax.axis_index("c")
    s = lax.axis_index("s")
    for r, (xc, srcs, dsts) in enumerate(((xc_a, s_ab, d_ab),
                                          (xc_b, s_bp, d_bp),
                                          (xc_p, s_pa, d_pa))):
        pltpu.sync_copy(zf_h, acc.at[pl.ds(s * RPT, RPT)])
        pltpu.sync_copy(srcs.at[c, s], isrc)
        pltpu.sync_copy(dsts.at[s], idst)
        plsc.subcore_barrier()

        def chunk(k, carry):
            pltpu.async_copy(xc.at[isrc.at[k]], gbuf, gsem).wait()
            pltpu.sync_copy(gbuf, acc.at[idst.at[k]], add=True)
            return carry

        lax.fori_loop(0, NK, chunk, 0)
        plsc.subcore_barrier()
        pltpu.sync_copy(acc.at[pl.ds(s * RPT, RPT)],
                        sums.at[r, c, pl.ds(s * RPT, RPT)])


def _cnt_body(d_ab, d_bp, d_pa, zn_h, cnts, cnt1d, idst):
    c = lax.axis_index("c")
    s = lax.axis_index("s")
    for r, dsts in enumerate((d_ab, d_bp, d_pa)):
        pltpu.sync_copy(zn_h, cnt1d)
        pltpu.sync_copy(dsts.at[s, pl.ds(c * NKH, NKH)], idst)

        def ebody(e, carry):
            k = e // (CH // 16)
            i = e % (CH // 16)
            idx = idst[k, pl.ds(i * 16, 16)]
            plsc.addupdate_scatter(cnt1d, [idx],
                                   jnp.full((16,), 1.0, jnp.float32))
            return carry

        lax.fori_loop(0, NKH * (CH // 16), ebody, 0)
        pltpu.sync_copy(cnt1d, cnts.at[r, c, s])


def _sc_mesh():
    return plsc.VectorSubcoreMesh(core_axis_name="c", subcore_axis_name="s",
                                  num_cores=NC, num_subcores=NS)


def _make_sum_call():
    return pl.kernel(
        _sum_body,
        out_type=jax.ShapeDtypeStruct((3, NC, NP, HALF), jnp.float32),
        mesh=_sc_mesh(),
        scratch_types=[
            pltpu.VMEM_SHARED((NP, HALF), jnp.float32),
            pltpu.VMEM((NK, CH), jnp.int32),
            pltpu.VMEM((NK, CH), jnp.int32),
            pltpu.VMEM((CH, HALF), jnp.float32),
            pltpu.SemaphoreType.DMA,
        ],
    )


def _make_cnt_call():
    return pl.kernel(
        _cnt_body,
        out_type=jax.ShapeDtypeStruct((3, NC, NS, NP), jnp.float32),
        mesh=_sc_mesh(),
        scratch_types=[
            pltpu.VMEM((NP,), jnp.float32),
            pltpu.VMEM((NKH, CH), jnp.int32),
        ],
        compiler_params=pltpu.CompilerParams(needs_layout_passes=False),
    )


BN = 2048
NB = NP // BN


def _tc_body(sums_ref, cnts_ref, x_ref, wl_ref, wr_ref, bl_ref, lw_ref,
             lb_ref, out_ref, p_scr, q_scr, r_scr):
    nb = pl.program_id(1)

    @pl.when(nb == 0)
    def _():
        lw = lw_ref[...]
        p_scr[...] = lax.dot_general(
            wl_ref[0], lw, (((0,), (1,)), ((), ())),
            preferred_element_type=jnp.float32)
        q_scr[...] = lax.dot_general(
            wr_ref[0], lw, (((0,), (1,)), ((), ())),
            preferred_element_type=jnp.float32)
        r_scr[...] = lax.dot_general(
            bl_ref[0], lw, (((1,), (1,)), ((), ())),
            preferred_element_type=jnp.float32) + lb_ref[...]

    sm = sums_ref[0]
    cnt = jnp.sum(cnts_ref[0], axis=(0, 1))[:, None]
    cl = jnp.maximum(cnt, 1.0)
    m0 = sm[0] / cl
    m1 = sm[1] / cl
    p = p_scr[...]
    h = (jnp.dot(m0, p[:HALF], preferred_element_type=jnp.float32)
         + jnp.dot(m1, p[HALF:], preferred_element_type=jnp.float32)
         + jnp.dot(x_ref[0], q_scr[...], preferred_element_type=jnp.float32)
         + r_scr[...])
    out_ref[0] = jnp.maximum(h, 0.0)


_tc_call = pl.pallas_call(
    _tc_body,
    grid=(3, NB),
    in_specs=[
        pl.BlockSpec((1, NC, BN, HALF), lambda t, b: (t, 0, b, 0)),
        pl.BlockSpec((1, NC, NS, BN), lambda t, b: (t, 0, 0, b)),
        pl.BlockSpec((1, BN, D), lambda t, b: (t, b, 0)),
        pl.BlockSpec((1, H, D), lambda t, b: (t, 0, 0)),
        pl.BlockSpec((1, H, D), lambda t, b: (t, 0, 0)),
        pl.BlockSpec((1, 1, H), lambda t, b: (t, 0, 0)),
        pl.BlockSpec((O, H), lambda t, b: (0, 0)),
        pl.BlockSpec((1, O), lambda t, b: (0, 0)),
    ],
    out_specs=pl.BlockSpec((1, BN, O), lambda t, b: (t, b, 0)),
    out_shape=jax.ShapeDtypeStruct((3, NP, O), jnp.float32),
    scratch_shapes=[pltpu.VMEM((D, O), jnp.float32),
                    pltpu.VMEM((D, O), jnp.float32),
                    pltpu.VMEM((1, O), jnp.float32)],
    compiler_params=pltpu.CompilerParams(
        dimension_semantics=("arbitrary", "arbitrary")),
)


def _prep(x, ei):
    xc = x.reshape(N, 2, HALF).transpose(1, 0, 2).reshape(2 * N, HALF)
    pad = EP - E
    src = jnp.concatenate([ei[0], jnp.zeros((pad,), jnp.int32)])
    dst = jnp.concatenate([ei[1], jnp.full((pad,), DUMP, jnp.int32)])
    src = src.reshape(NS, NK, CH)
    dst = dst.reshape(NS, NK, CH)
    srcs = jnp.stack([src, src + N])
    return xc, srcs, dst


def kernel(x_adresse, x_batiment, x_parcelle, ei_ab, ei_bp, ei_pa,
           Wl_ab, bl_ab, Wr_ab, Wl_bp, bl_bp, Wr_bp, Wl_pa, bl_pa, Wr_pa,
           lin_W, lin_b):
    xc_a, s_ab, d_ab = _prep(x_adresse, ei_ab)
    xc_b, s_bp, d_bp = _prep(x_batiment, ei_bp)
    xc_p, s_pa, d_pa = _prep(x_parcelle, ei_pa)
    zf = jnp.zeros((RPT, HALF), jnp.float32)
    zn = jnp.zeros((NP,), jnp.float32)
    sums = _make_sum_call()(xc_a, xc_b, xc_p, s_ab, d_ab, s_bp, d_bp,
                            s_pa, d_pa, zf)
    cnts = _make_cnt_call()(d_ab, d_bp, d_pa, zn)
    x_all = jnp.pad(jnp.stack([x_batiment, x_parcelle, x_adresse]),
                    ((0, 0), (0, NP - N), (0, 0)))
    wl = jnp.stack([Wl_ab, Wl_bp, Wl_pa])
    wr = jnp.stack([Wr_ab, Wr_bp, Wr_pa])
    bl = jnp.stack([bl_ab, bl_bp, bl_pa]).reshape(3, 1, H)
    out = _tc_call(sums, cnts, x_all, wl, wr, bl, lin_W,
                   lin_b.reshape(1, O))
    return (out[2, :N], out[0, :N], out[1, :N])
